# baseline (speedup 1.0000x reference)
import jax
import jax.numpy as jnp
from jax.experimental import pallas as pl
from jax.experimental.pallas import tpu as pltpu

_BSZ, _SEQ, _DIN, _DOUT, _E, _K, _R = 4, 2048, 768, 768, 64, 8, 16
_NB = 2
_SB = _SEQ // _NB


def _moelora_body(idx_ref, attn_ref, x_ref, wt_ref, b_ref, ap_ref, bp_ref,
                  bias_ref, out_ref):
    bi = pl.program_id(0)
    bias_acc = b_ref[:]
    delta_t = None
    for k in range(_K):
        e = idx_ref[bi, k]
        w = attn_ref[bi, k]
        a_k = ap_ref[pl.ds(e, 1)].reshape(_DIN, _R).astype(jnp.bfloat16)
        b_k = (bp_ref[pl.ds(e, 1)].reshape(_R, _DOUT) * w).astype(jnp.bfloat16)
        part = jax.lax.dot_general(
            b_k, a_k, (((0,), (1,)), ((), ())),
            preferred_element_type=jnp.float32)
        delta_t = part if delta_t is None else delta_t + part
        bias_acc = bias_acc + w * bias_ref[pl.ds(e, 1), :]
    m_t = (wt_ref[:] + delta_t).astype(jnp.bfloat16)
    out_ref[0] = jax.lax.dot_general(
        x_ref[0].astype(jnp.bfloat16), m_t,
        (((1,), (1,)), ((), ())),
        preferred_element_type=jnp.float32) + bias_acc


@jax.jit
def _run(x, attn, idx, w, b2, ap, bp, bias_pool):
    return pl.pallas_call(
        _moelora_body,
        grid=(_BSZ, _NB),
        in_specs=[
            pl.BlockSpec(memory_space=pltpu.SMEM),
            pl.BlockSpec(memory_space=pltpu.SMEM),
            pl.BlockSpec((1, _SB, _DIN), lambda i, j: (i, j, 0)),
            pl.BlockSpec((_DOUT, _DIN), lambda i, j: (0, 0)),
            pl.BlockSpec((1, _DOUT), lambda i, j: (0, 0)),
            pl.BlockSpec((_E, _DIN, _R), lambda i, j: (0, 0, 0)),
            pl.BlockSpec((_E, _R, _DOUT), lambda i, j: (0, 0, 0)),
            pl.BlockSpec((_E, _DOUT), lambda i, j: (0, 0)),
        ],
        out_specs=pl.BlockSpec((1, _SB, _DOUT), lambda i, j: (i, j, 0)),
        out_shape=jax.ShapeDtypeStruct((_BSZ, _SEQ, _DOUT), jnp.float32),
    )(idx, attn, x, w, b2, ap, bp, bias_pool)


def kernel(x, topk_attn, topk_idx, W, b, A_pool, B_pool, bias_pool):
    b2 = b.reshape(1, _DOUT)
    idx = topk_idx.astype(jnp.int32)
    return _run(x, topk_attn, idx, W, b2, A_pool, B_pool, bias_pool)


# speedup vs baseline: 2.8024x; 2.8024x over previous
import jax
import jax.numpy as jnp
from jax.experimental import pallas as pl
from jax.experimental.pallas import tpu as pltpu

_BSZ, _SEQ, _DIN, _DOUT, _E, _K, _R = 4, 2048, 768, 768, 64, 8, 16


def _moelora_body(idx_ref, attn_ref, x_ref, wt_ref, b_ref, apt_ref, bp_ref,
                  bias_ref, out_ref):
    bi = pl.program_id(0)
    a_parts = []
    b_parts = []
    bias_acc = b_ref[:]
    for k in range(_K):
        e = idx_ref[bi, k]
        w = attn_ref[bi, k]
        a_parts.append(apt_ref[pl.ds(e, 1)].reshape(_R, _DIN))
        b_parts.append(bp_ref[pl.ds(e, 1)].reshape(_R, _DOUT) * w)
        bias_acc = bias_acc + w * bias_ref[pl.ds(e, 1), :]
    acat_t = jnp.concatenate(a_parts, axis=0)
    bcat = jnp.concatenate(b_parts, axis=0)
    delta_t = jax.lax.dot_general(
        bcat.astype(jnp.bfloat16), acat_t.astype(jnp.bfloat16),
        (((0,), (0,)), ((), ())),
        preferred_element_type=jnp.float32)
    m_t = (wt_ref[:] + delta_t).astype(jnp.bfloat16)
    out_ref[0] = jax.lax.dot_general(
        x_ref[0].astype(jnp.bfloat16), m_t,
        (((1,), (1,)), ((), ())),
        preferred_element_type=jnp.float32) + bias_acc


@jax.jit
def _run(x, attn, idx, wt, b2, apt, bp, bias_pool):
    return pl.pallas_call(
        _moelora_body,
        grid=(_BSZ,),
        in_specs=[
            pl.BlockSpec(memory_space=pltpu.SMEM),
            pl.BlockSpec(memory_space=pltpu.SMEM),
            pl.BlockSpec((1, _SEQ, _DIN), lambda i: (i, 0, 0)),
            pl.BlockSpec((_DOUT, _DIN), lambda i: (0, 0)),
            pl.BlockSpec((1, _DOUT), lambda i: (0, 0)),
            pl.BlockSpec((_E, _R, _DIN), lambda i: (0, 0, 0)),
            pl.BlockSpec((_E, _R, _DOUT), lambda i: (0, 0, 0)),
            pl.BlockSpec((_E, _DOUT), lambda i: (0, 0)),
        ],
        out_specs=pl.BlockSpec((1, _SEQ, _DOUT), lambda i: (i, 0, 0)),
        out_shape=jax.ShapeDtypeStruct((_BSZ, _SEQ, _DOUT), jnp.float32),
    )(idx, attn, x, wt, b2, apt, bp, bias_pool)


def kernel(x, topk_attn, topk_idx, W, b, A_pool, B_pool, bias_pool):
    apt = A_pool.transpose(0, 2, 1)
    b2 = b.reshape(1, _DOUT)
    idx = topk_idx.astype(jnp.int32)
    return _run(x, topk_attn, idx, W, b2, apt, B_pool, bias_pool)
